# baseline (speedup 1.0000x reference)
import functools

import numpy as np
import jax
import jax.numpy as jnp
from jax import lax
from jax.experimental import pallas as pl
from jax.experimental.pallas import tpu as pltpu
from jax.experimental.pallas import tpu_sc as plsc

_VOCAB = 1000000
_D = 64
_MAXLEN = 200
_B = 4096
_L = 200

_NC = 2
_NS = 16
_NW = _NC * _NS
_TB = 128
_TASKS = _L * (_B // _TB)
_TPW = _TASKS // _NW
_ROWS = _B * _L
_NBUF = 4
_LOOK = 2
_UNROLL = 4


def _pe_table(maxlen, d):
    pos = np.arange(maxlen, dtype=np.float32)[:, None]
    i = np.arange(d, dtype=np.float32)[None, :]
    angle_rates = 1.0 / np.power(10000.0, (2.0 * np.floor(i / 2.0)) / float(d))
    angles = pos * angle_rates
    pe = np.zeros((maxlen, d), dtype=np.float32)
    pe[:, 0::2] = np.sin(angles[:, 0::2])
    pe[:, 1::2] = np.cos(angles[:, 1::2])
    return pe


_PE_TABLE = _pe_table(_MAXLEN, _D)


def _sc_body(x2, table, pe, out, idx_v, idxw, rows, pe_v, gsems, osems):
    wid = lax.axis_index("s") * _NC + lax.axis_index("c")
    u0 = wid * _TPW
    pltpu.sync_copy(x2.at[pl.ds(u0, _TPW)], idx_v)
    pltpu.sync_copy(pe, pe_v)
    iota16 = lax.iota(jnp.int32, 16)

    def task_geom(c):
        u = u0 + c
        l = lax.shift_right_logical(u, 8) * 8 + lax.bitwise_and(u, 7)
        bt = lax.bitwise_and(lax.shift_right_logical(u, 3), 31)
        return l, bt * _TB * _MAXLEN + l

    def gather_issue(c, rb):
        pltpu.async_copy(table.at[idx_v.at[c]], rows[rb], gsems[rb])

    def gather_wait(c, rb):
        pltpu.make_async_copy(table.at[idx_v.at[c]], rows[rb], gsems[rb]).wait()

    def add_pe_build_idx(c, rb):
        l, wbase = task_geom(c)
        pv = [pe_v[l, pl.ds(16 * q, 16)] for q in range(_D // 16)]
        for k in range(_TB // 16):
            idxw[rb, pl.ds(16 * k, 16)] = (iota16 + 16 * k) * _MAXLEN + wbase

        def row_body(r4, carry):
            base = r4 * _UNROLL
            for u in range(_UNROLL):
                r = base + u
                for q in range(_D // 16):
                    plsc.addupdate(rows[rb].at[r, pl.ds(16 * q, 16)], pv[q])
            return carry

        lax.fori_loop(0, _TB // _UNROLL, row_body, 0)

    def scat_issue(c, rb):
        pltpu.async_copy(rows[rb], out.at[idxw.at[rb]], osems[rb])

    def scat_wait(c, rb):
        pltpu.make_async_copy(rows[rb], out.at[idxw.at[rb]], osems[rb]).wait()

    def consume(c, rb):
        gather_wait(c, rb)
        add_pe_build_idx(c, rb)
        scat_issue(c, rb)

    gather_issue(0, 0)
    gather_issue(1, 1)
    consume(0, 0)
    gather_issue(2, 2)
    consume(1, 1)
    gather_issue(3, 3)

    def group_body(g, carry):
        for i4 in range(_NBUF):
            c = 2 + g * _NBUF + i4
            rb = (2 + i4) % _NBUF
            bn = i4 % _NBUF
            consume(c, rb)
            scat_wait(c - 2, bn)
            gather_issue(c + _LOOK, bn)
        return carry

    lax.fori_loop(0, (_TPW - _LOOK - 2) // _NBUF, group_body, 0)

    consume(_TPW - 2, (_TPW - 2) % _NBUF)
    consume(_TPW - 1, (_TPW - 1) % _NBUF)
    for c in range(_TPW - _NBUF, _TPW):
        scat_wait(c, c % _NBUF)


_sc_embed = pl.kernel(
    _sc_body,
    out_type=jax.ShapeDtypeStruct((_ROWS, _D), jnp.float32),
    mesh=plsc.VectorSubcoreMesh(core_axis_name="c", subcore_axis_name="s"),
    compiler_params=pltpu.CompilerParams(use_tc_tiling_on_sc=False),
    scratch_types=[
        pltpu.VMEM((_TPW, _TB), jnp.int32),
        pltpu.VMEM((_NBUF, _TB), jnp.int32),
        [pltpu.VMEM((_TB, _D), jnp.float32) for _ in range(_NBUF)],
        pltpu.VMEM((_MAXLEN, _D), jnp.float32),
        [pltpu.SemaphoreType.DMA for _ in range(_NBUF)],
        [pltpu.SemaphoreType.DMA for _ in range(_NBUF)],
    ],
)


def kernel(x, W):
    x2 = x.reshape(32, 128, 25, 8).transpose(2, 0, 3, 1).reshape(_TASKS, _TB)
    out = _sc_embed(x2, W, jnp.asarray(_PE_TABLE))
    return out.reshape(_B, _L, _D)
